# baseline (speedup 1.0000x reference)
import jax
import jax.numpy as jnp
from jax.experimental import pallas as pl

N = 10000
DEG = 16
E = N * DEG
H = 256
SB = 16
NG = 64
BN = 400
GRID = N // BN
KK = DEG + 1


def _fold_kernel(emb_edge_ref, emb_static_ref, emb_r_ref, wc_ref, wev_ref, wek_ref,
                 tv1_ref, tv2_ref, tv3_ref, trk_ref):
    wc = wc_ref[...]
    wev = wev_ref[...]
    ee = emb_edge_ref[...]
    es = emb_static_ref[...]
    f32 = jnp.float32
    tv1_ref[...] = jnp.dot(jnp.dot(ee, wc[0:H, :], preferred_element_type=f32),
                           wev, preferred_element_type=f32)
    tv2_ref[...] = jnp.dot(jnp.dot(ee, wc[H:2 * H, :], preferred_element_type=f32),
                           wev, preferred_element_type=f32)
    tv3_ref[...] = jnp.dot(jnp.dot(es, wc[2 * H:3 * H, :], preferred_element_type=f32),
                           wev, preferred_element_type=f32)
    trk_ref[...] = jnp.dot(emb_r_ref[...], wek_ref[...], preferred_element_type=f32)


def _node_kernel(idxv_ref, emb_v_ref, wq_ref, wk_ref, wv_ref, wg1_ref, bg1_ref,
                 wg2_ref, bg2_ref, nf_ref, q_ref, kn_ref, vn_ref, su_ref):
    f32 = jnp.float32
    idx = idxv_ref[...]
    oh = (idx == jax.lax.broadcasted_iota(jnp.int32, (BN, SB), 1)).astype(f32)
    nf = jnp.dot(oh, emb_v_ref[...], preferred_element_type=f32)
    nf_ref[...] = nf
    q_ref[...] = jnp.dot(nf, wq_ref[...], preferred_element_type=f32)
    kn_ref[...] = jnp.dot(nf, wk_ref[...], preferred_element_type=f32)
    vn_ref[...] = jnp.dot(nf, wv_ref[...], preferred_element_type=f32)
    h1 = jax.nn.relu(jnp.dot(nf, wg1_ref[...], preferred_element_type=f32)
                     + bg1_ref[...])
    su_ref[...] = jax.nn.sigmoid(jnp.dot(h1, wg2_ref[...], preferred_element_type=f32)
                                 + bg2_ref[...])


def _attn_kernel(eidx_ref, eidxrev_ref, state_ref, scal_ref, sender_ref, nscal_ref,
                 nf_ref, q_ref, kn_ref, vn_ref, su_ref, ksrc_ref, vsrc_ref,
                 tv1_ref, tv2_ref, tv3_ref, trk_ref, emb_edge_ref,
                 nout_ref, eout_ref):
    f32 = jnp.float32
    i32 = jnp.int32
    q = q_ref[...]
    kn = kn_ref[...]
    vn = vn_ref[...]
    nf = nf_ref[...]
    su = su_ref[...]
    nscal = nscal_ref[...]
    scal = scal_ref[...]
    sender = sender_ref[...]
    state = state_ref[...]
    eidx = eidx_ref[...]
    eidxrev = eidxrev_ref[...]
    tv1 = tv1_ref[...]
    tv2 = tv2_ref[...]
    tv3 = tv3_ref[...]
    trk = trk_ref[...]
    emb_edge = emb_edge_ref[...]

    iota16 = jax.lax.broadcasted_iota(i32, (BN, SB), 1)
    iota4 = jax.lax.broadcasted_iota(i32, (BN, 4), 1)

    r1 = (scal < nscal).astype(i32)
    r2 = ((sender + scal) < nscal).astype(i32)
    fidx = r1 + 2 * r2

    logit_cols = [jnp.sum(q * kn, axis=1, keepdims=True) * (1.0 / 16.0)]
    v_list = []
    ef_list = []
    for d in range(DEG):
        sc_d = scal[:, d:d + 1]
        st_d = state[:, d:d + 1]
        eq = (state == st_d)
        minv = jnp.min(jnp.where(eq, scal, 1e9), axis=1, keepdims=True)
        best = (sc_d == minv).astype(i32)
        oh_r = (st_d + best == iota16).astype(f32)
        oh_e = (eidx[:, d:d + 1] == iota16).astype(f32)
        oh_rev = (eidxrev[:, d:d + 1] == iota16).astype(f32)
        oh_f = (fidx[:, d:d + 1] == iota4).astype(f32)
        ef_d = jnp.dot(oh_e, emb_edge, preferred_element_type=f32)
        ev_d = (jnp.dot(oh_e, tv1, preferred_element_type=f32)
                + jnp.dot(oh_rev, tv2, preferred_element_type=f32)
                + jnp.dot(oh_f, tv3, preferred_element_type=f32))
        ek_d = jnp.dot(oh_r, trk, preferred_element_type=f32)
        k_d = ksrc_ref[:, d * H:(d + 1) * H] + ek_d
        v_d = vsrc_ref[:, d * H:(d + 1) * H] + ev_d
        logit_cols.append(jnp.sum(q * k_d, axis=1, keepdims=True) * (1.0 / 16.0))
        v_list.append(v_d)
        ef_list.append(ef_d)

    logits = jnp.concatenate(logit_cols, axis=1)

    col = jax.lax.broadcasted_iota(i32, (BN, KK), 1)
    colf = col.astype(f32)
    ranks = jnp.zeros((BN, KK), dtype=i32)
    for j in range(KK):
        xj = logits[:, j:j + 1]
        ranks = ranks + (xj > logits).astype(i32) \
            + ((xj == logits) & (j < col)).astype(i32)

    lsq = logits * logits
    cz_cols = []
    cz2_cols = []
    s_cols = []
    for k in range(KK):
        le = (ranks <= k)
        cz_cols.append(jnp.sum(jnp.where(le, logits, 0.0), axis=1, keepdims=True))
        cz2_cols.append(jnp.sum(jnp.where(le, lsq, 0.0), axis=1, keepdims=True))
        eqk = (ranks == k)
        s_cols.append(jnp.sum(jnp.where(eqk, logits, 0.0), axis=1, keepdims=True))
    cz = jnp.concatenate(cz_cols, axis=1)
    cz2 = jnp.concatenate(cz2_cols, axis=1)
    s = jnp.concatenate(s_cols, axis=1)

    kr = colf + 1.0
    mz = cz / kr
    mz2 = cz2 / kr
    discr = jax.nn.relu(mz * mz - mz2 + 1.0 / kr)
    tau_c = mz - jnp.sqrt(discr + 1e-12)
    kidx15 = jnp.sum((s > tau_c).astype(f32), axis=1, keepdims=True)
    tau15 = jnp.sum(jnp.where(colf == (kidx15 - 1.0), tau_c, 0.0),
                    axis=1, keepdims=True)
    t15 = jax.nn.relu(logits - tau15)
    p15 = t15 * t15
    kidxsp = jnp.sum((kr * s > cz - 1.0).astype(f32), axis=1, keepdims=True)
    ck = jnp.sum(jnp.where(colf == (kidxsp - 1.0), cz, 0.0), axis=1, keepdims=True)
    tausp = (ck - 1.0) / kidxsp
    psp = jax.nn.relu(logits - tausp)
    m = jnp.max(logits, axis=1, keepdims=True)
    ex = jnp.exp(logits - m)
    psoft = ex / jnp.sum(ex, axis=1, keepdims=True)

    w_low = su * 2.0
    w_high = (su - 0.5) * 2.0
    probs_low = (1.0 - w_low) * psoft + w_low * p15
    probs_high = (1.0 - w_high) * p15 + w_high * psp
    probs = jnp.where(su <= 0.5, probs_low, probs_high)

    is_sel = (probs > 1e-06).astype(f32)
    nsel = jnp.sum(is_sel, axis=1, keepdims=True)
    attn = is_sel / (nsel + 1e-09)

    agg = attn[:, 0:1] * vn
    for d in range(DEG):
        agg = agg + attn[:, d + 1:d + 2] * v_list[d]

    nout_ref[...] = nf + agg
    for d in range(DEG):
        eout_ref[:, d * H:(d + 1) * H] = ef_list[d] + agg


def kernel(node_states, edge_states, scalars, edge_index, batch, batched_reverse_idx,
           training_step, emb_v, emb_r, emb_edge, emb_static, Wq, Wk, Wv, Wek, Wev,
           Wc, Wg1, bg1, Wg2, bg2):
    f32 = jnp.float32
    i32 = jnp.int32
    src = edge_index[0]
    node_states = node_states.astype(i32)
    edge_states = edge_states.astype(i32)
    batch = batch.astype(i32)
    rev = batched_reverse_idx.astype(i32)

    w3 = (2 ** jnp.arange(3, dtype=i32))
    w4 = (2 ** jnp.arange(4, dtype=i32))
    s2 = 2 * jnp.sum(node_states * w3, axis=-1)
    node_scal = scalars[0::DEG, 0]
    seg = batch * SB + s2
    segmin = jax.ops.segment_min(node_scal, seg, num_segments=NG * SB)
    best_n = (node_scal == segmin[seg]).astype(i32)
    idx_v = (s2 + best_n).reshape(N, 1)

    eidx = jnp.sum(edge_states * w4, axis=-1)
    eidx_rev = eidx[rev]
    state_e = s2[src]
    sender = node_scal[src]

    eidx_g = eidx.reshape(N, DEG)
    eidxrev_g = eidx_rev.reshape(N, DEG)
    state_g = state_e.reshape(N, DEG)
    scal_g = scalars[:, 0].reshape(N, DEG)
    sender_g = sender.reshape(N, DEG)
    nscal = node_scal.reshape(N, 1)

    bfull = lambda shp: pl.BlockSpec(shp, lambda i: tuple(0 for _ in shp))
    bnode = lambda w: pl.BlockSpec((BN, w), lambda i: (i, 0))

    tv1, tv2, tv3, trk = pl.pallas_call(
        _fold_kernel,
        grid=(1,),
        in_specs=[bfull((SB, H)), bfull((4, H)), bfull((SB, H)),
                  bfull((3 * H, H)), bfull((H, H)), bfull((H, H))],
        out_specs=[bfull((SB, H)), bfull((SB, H)), bfull((4, H)), bfull((SB, H))],
        out_shape=[jax.ShapeDtypeStruct((SB, H), f32),
                   jax.ShapeDtypeStruct((SB, H), f32),
                   jax.ShapeDtypeStruct((4, H), f32),
                   jax.ShapeDtypeStruct((SB, H), f32)],
    )(emb_edge, emb_static, emb_r, Wc, Wev, Wek)

    nf, q, kn, vn, su = pl.pallas_call(
        _node_kernel,
        grid=(GRID,),
        in_specs=[bnode(1), bfull((SB, H)), bfull((H, H)), bfull((H, H)),
                  bfull((H, H)), bfull((H, H)), bfull((1, H)),
                  bfull((H, 1)), bfull((1, 1))],
        out_specs=[bnode(H), bnode(H), bnode(H), bnode(H), bnode(1)],
        out_shape=[jax.ShapeDtypeStruct((N, H), f32)] * 4
        + [jax.ShapeDtypeStruct((N, 1), f32)],
    )(idx_v, emb_v, Wq, Wk, Wv, Wg1, bg1.reshape(1, H), Wg2, bg2.reshape(1, 1))

    ksrc = kn[src].reshape(N, DEG * H)
    vsrc = vn[src].reshape(N, DEG * H)

    bedge = pl.BlockSpec((BN, DEG * H), lambda i: (i, 0))
    nout, eout_t = pl.pallas_call(
        _attn_kernel,
        grid=(GRID,),
        in_specs=[bnode(DEG), bnode(DEG), bnode(DEG), bnode(DEG), bnode(DEG),
                  bnode(1), bnode(H), bnode(H), bnode(H), bnode(H), bnode(1),
                  bedge, bedge,
                  bfull((SB, H)), bfull((SB, H)), bfull((4, H)), bfull((SB, H)),
                  bfull((SB, H))],
        out_specs=[bnode(H), bedge],
        out_shape=[jax.ShapeDtypeStruct((N, H), f32),
                   jax.ShapeDtypeStruct((N, DEG * H), f32)],
    )(eidx_g, eidxrev_g, state_g, scal_g, sender_g, nscal,
      nf, q, kn, vn, su, ksrc, vsrc, tv1, tv2, tv3, trk, emb_edge)

    edge_out = eout_t.reshape(E, H)
    return (nout, edge_out)


# speedup vs baseline: 1.0610x; 1.0610x over previous
import jax
import jax.numpy as jnp
from jax.experimental import pallas as pl

N = 10000
DEG = 16
E = N * DEG
H = 256
SB = 16
NG = 64
BN = 400
GRID = N // BN
KK = DEG + 1


def _fold_kernel(emb_v_ref, emb_edge_ref, emb_static_ref, emb_r_ref,
                 wq_ref, wk_ref, wv_ref, wc_ref, wev_ref, wek_ref,
                 wg1_ref, bg1_ref, wg2_ref, bg2_ref,
                 tq_ref, tk_ref, tv_ref, su_ref,
                 tv1_ref, tv2_ref, tv3_ref, trk_ref):
    f32 = jnp.float32
    ev = emb_v_ref[...]
    wc = wc_ref[...]
    wev = wev_ref[...]
    ee = emb_edge_ref[...]
    es = emb_static_ref[...]
    tq_ref[...] = jnp.dot(ev, wq_ref[...], preferred_element_type=f32)
    tk_ref[...] = jnp.dot(ev, wk_ref[...], preferred_element_type=f32)
    tv_ref[...] = jnp.dot(ev, wv_ref[...], preferred_element_type=f32)
    h1 = jax.nn.relu(jnp.dot(ev, wg1_ref[...], preferred_element_type=f32)
                     + bg1_ref[...])
    su_ref[...] = jax.nn.sigmoid(jnp.dot(h1, wg2_ref[...], preferred_element_type=f32)
                                 + bg2_ref[...])
    tv1_ref[...] = jnp.dot(jnp.dot(ee, wc[0:H, :], preferred_element_type=f32),
                           wev, preferred_element_type=f32)
    tv2_ref[...] = jnp.dot(jnp.dot(ee, wc[H:2 * H, :], preferred_element_type=f32),
                           wev, preferred_element_type=f32)
    tv3_ref[...] = jnp.dot(jnp.dot(es, wc[2 * H:3 * H, :], preferred_element_type=f32),
                           wev, preferred_element_type=f32)
    trk_ref[...] = jnp.dot(emb_r_ref[...], wek_ref[...], preferred_element_type=f32)


def _attn_kernel(idxv_ref, idxvsrc_ref, eidx_ref, eidxrev_ref, state_ref,
                 scal_ref, sender_ref, nscal_ref,
                 emb_v_ref, tq_ref, tk_ref, tv_ref, su16_ref,
                 tv1_ref, tv2_ref, tv3_ref, trk_ref, emb_edge_ref,
                 nout_ref, eout_ref):
    f32 = jnp.float32
    i32 = jnp.int32
    nscal = nscal_ref[...]
    scal = scal_ref[...]
    sender = sender_ref[...]
    state = state_ref[...]
    eidx = eidx_ref[...]
    eidxrev = eidxrev_ref[...]
    idxvsrc = idxvsrc_ref[...]
    tq = tq_ref[...]
    tk = tk_ref[...]
    tv = tv_ref[...]
    su16 = su16_ref[...]
    tv1 = tv1_ref[...]
    tv2 = tv2_ref[...]
    tv3 = tv3_ref[...]
    trk = trk_ref[...]
    emb_edge = emb_edge_ref[...]

    iota16 = jax.lax.broadcasted_iota(i32, (BN, SB), 1)
    iota4 = jax.lax.broadcasted_iota(i32, (BN, 4), 1)

    oh_n = (idxv_ref[...] == iota16).astype(f32)
    nf = jnp.dot(oh_n, emb_v_ref[...], preferred_element_type=f32)
    q = jnp.dot(oh_n, tq, preferred_element_type=f32)
    kn = jnp.dot(oh_n, tk, preferred_element_type=f32)
    vn = jnp.dot(oh_n, tv, preferred_element_type=f32)
    su = jnp.dot(oh_n, su16, preferred_element_type=f32)

    r1 = (scal < nscal).astype(i32)
    r2 = ((sender + scal) < nscal).astype(i32)
    fidx = r1 + 2 * r2

    logit_cols = [jnp.sum(q * kn, axis=1, keepdims=True) * (1.0 / 16.0)]
    v_list = []
    ef_list = []
    for d in range(DEG):
        sc_d = scal[:, d:d + 1]
        st_d = state[:, d:d + 1]
        eq = (state == st_d)
        minv = jnp.min(jnp.where(eq, scal, 1e9), axis=1, keepdims=True)
        best = (sc_d == minv).astype(i32)
        oh_r = (st_d + best == iota16).astype(f32)
        oh_e = (eidx[:, d:d + 1] == iota16).astype(f32)
        oh_rev = (eidxrev[:, d:d + 1] == iota16).astype(f32)
        oh_f = (fidx[:, d:d + 1] == iota4).astype(f32)
        oh_s = (idxvsrc[:, d:d + 1] == iota16).astype(f32)
        ef_d = jnp.dot(oh_e, emb_edge, preferred_element_type=f32)
        ev_d = (jnp.dot(oh_e, tv1, preferred_element_type=f32)
                + jnp.dot(oh_rev, tv2, preferred_element_type=f32)
                + jnp.dot(oh_f, tv3, preferred_element_type=f32))
        ek_d = jnp.dot(oh_r, trk, preferred_element_type=f32)
        k_d = jnp.dot(oh_s, tk, preferred_element_type=f32) + ek_d
        v_d = jnp.dot(oh_s, tv, preferred_element_type=f32) + ev_d
        logit_cols.append(jnp.sum(q * k_d, axis=1, keepdims=True) * (1.0 / 16.0))
        v_list.append(v_d)
        ef_list.append(ef_d)

    logits = jnp.concatenate(logit_cols, axis=1)

    col = jax.lax.broadcasted_iota(i32, (BN, KK), 1)
    colf = col.astype(f32)
    ranks = jnp.zeros((BN, KK), dtype=i32)
    for j in range(KK):
        xj = logits[:, j:j + 1]
        ranks = ranks + (xj > logits).astype(i32) \
            + ((xj == logits) & (j < col)).astype(i32)

    lsq = logits * logits
    cz_cols = []
    cz2_cols = []
    s_cols = []
    for k in range(KK):
        le = (ranks <= k)
        cz_cols.append(jnp.sum(jnp.where(le, logits, 0.0), axis=1, keepdims=True))
        cz2_cols.append(jnp.sum(jnp.where(le, lsq, 0.0), axis=1, keepdims=True))
        eqk = (ranks == k)
        s_cols.append(jnp.sum(jnp.where(eqk, logits, 0.0), axis=1, keepdims=True))
    cz = jnp.concatenate(cz_cols, axis=1)
    cz2 = jnp.concatenate(cz2_cols, axis=1)
    s = jnp.concatenate(s_cols, axis=1)

    kr = colf + 1.0
    mz = cz / kr
    mz2 = cz2 / kr
    discr = jax.nn.relu(mz * mz - mz2 + 1.0 / kr)
    tau_c = mz - jnp.sqrt(discr + 1e-12)
    kidx15 = jnp.sum((s > tau_c).astype(f32), axis=1, keepdims=True)
    tau15 = jnp.sum(jnp.where(colf == (kidx15 - 1.0), tau_c, 0.0),
                    axis=1, keepdims=True)
    t15 = jax.nn.relu(logits - tau15)
    p15 = t15 * t15
    kidxsp = jnp.sum((kr * s > cz - 1.0).astype(f32), axis=1, keepdims=True)
    ck = jnp.sum(jnp.where(colf == (kidxsp - 1.0), cz, 0.0), axis=1, keepdims=True)
    tausp = (ck - 1.0) / kidxsp
    psp = jax.nn.relu(logits - tausp)
    m = jnp.max(logits, axis=1, keepdims=True)
    ex = jnp.exp(logits - m)
    psoft = ex / jnp.sum(ex, axis=1, keepdims=True)

    w_low = su * 2.0
    w_high = (su - 0.5) * 2.0
    probs_low = (1.0 - w_low) * psoft + w_low * p15
    probs_high = (1.0 - w_high) * p15 + w_high * psp
    probs = jnp.where(su <= 0.5, probs_low, probs_high)

    is_sel = (probs > 1e-06).astype(f32)
    nsel = jnp.sum(is_sel, axis=1, keepdims=True)
    attn = is_sel / (nsel + 1e-09)

    agg = attn[:, 0:1] * vn
    for d in range(DEG):
        agg = agg + attn[:, d + 1:d + 2] * v_list[d]

    nout_ref[...] = nf + agg
    for d in range(DEG):
        eout_ref[:, d * H:(d + 1) * H] = ef_list[d] + agg


def kernel(node_states, edge_states, scalars, edge_index, batch, batched_reverse_idx,
           training_step, emb_v, emb_r, emb_edge, emb_static, Wq, Wk, Wv, Wek, Wev,
           Wc, Wg1, bg1, Wg2, bg2):
    f32 = jnp.float32
    i32 = jnp.int32
    src = edge_index[0]
    node_states = node_states.astype(i32)
    edge_states = edge_states.astype(i32)
    batch = batch.astype(i32)
    rev = batched_reverse_idx.astype(i32)

    w3 = (2 ** jnp.arange(3, dtype=i32))
    w4 = (2 ** jnp.arange(4, dtype=i32))
    s2 = 2 * jnp.sum(node_states * w3, axis=-1)
    node_scal = scalars[0::DEG, 0]
    seg = batch * SB + s2
    segmin = jax.ops.segment_min(node_scal, seg, num_segments=NG * SB)
    best_n = (node_scal == segmin[seg]).astype(i32)
    idx_v = s2 + best_n

    eidx = jnp.sum(edge_states * w4, axis=-1)
    eidx_rev = eidx[rev]
    state_e = s2[src]
    sender = node_scal[src]
    idxv_src = idx_v[src]

    eidx_g = eidx.reshape(N, DEG)
    eidxrev_g = eidx_rev.reshape(N, DEG)
    state_g = state_e.reshape(N, DEG)
    scal_g = scalars[:, 0].reshape(N, DEG)
    sender_g = sender.reshape(N, DEG)
    idxvsrc_g = idxv_src.reshape(N, DEG)
    nscal = node_scal.reshape(N, 1)

    bfull = lambda shp: pl.BlockSpec(shp, lambda i: tuple(0 for _ in shp))
    bnode = lambda w: pl.BlockSpec((BN, w), lambda i: (i, 0))

    tq, tk, tv, su16, tv1, tv2, tv3, trk = pl.pallas_call(
        _fold_kernel,
        grid=(1,),
        in_specs=[bfull((SB, H)), bfull((SB, H)), bfull((4, H)), bfull((SB, H)),
                  bfull((H, H)), bfull((H, H)), bfull((H, H)),
                  bfull((3 * H, H)), bfull((H, H)), bfull((H, H)),
                  bfull((H, H)), bfull((1, H)), bfull((H, 1)), bfull((1, 1))],
        out_specs=[bfull((SB, H)), bfull((SB, H)), bfull((SB, H)), bfull((SB, 1)),
                   bfull((SB, H)), bfull((SB, H)), bfull((4, H)), bfull((SB, H))],
        out_shape=[jax.ShapeDtypeStruct((SB, H), f32)] * 3
        + [jax.ShapeDtypeStruct((SB, 1), f32)]
        + [jax.ShapeDtypeStruct((SB, H), f32),
           jax.ShapeDtypeStruct((SB, H), f32),
           jax.ShapeDtypeStruct((4, H), f32),
           jax.ShapeDtypeStruct((SB, H), f32)],
    )(emb_v, emb_edge, emb_static, emb_r, Wq, Wk, Wv, Wc, Wev, Wek,
      Wg1, bg1.reshape(1, H), Wg2, bg2.reshape(1, 1))

    bedge = pl.BlockSpec((BN, DEG * H), lambda i: (i, 0))
    nout, eout = pl.pallas_call(
        _attn_kernel,
        grid=(GRID,),
        in_specs=[bnode(1), bnode(DEG), bnode(DEG), bnode(DEG), bnode(DEG),
                  bnode(DEG), bnode(DEG), bnode(1),
                  bfull((SB, H)), bfull((SB, H)), bfull((SB, H)), bfull((SB, H)),
                  bfull((SB, 1)),
                  bfull((SB, H)), bfull((SB, H)), bfull((4, H)), bfull((SB, H)),
                  bfull((SB, H))],
        out_specs=[bnode(H), bedge],
        out_shape=[jax.ShapeDtypeStruct((N, H), f32),
                   jax.ShapeDtypeStruct((N, DEG * H), f32)],
    )(idx_v.reshape(N, 1), idxvsrc_g, eidx_g, eidxrev_g, state_g,
      scal_g, sender_g, nscal,
      emb_v, tq, tk, tv, su16, tv1, tv2, tv3, trk, emb_edge)

    edge_out = eout.reshape(E, H)
    return (nout, edge_out)
